# baseline (speedup 1.0000x reference)
import functools

import jax
import jax.numpy as jnp
from jax import lax
from jax.experimental import pallas as pl
from jax.experimental.pallas import tpu as pltpu
from jax.experimental.pallas import tpu_sc as plsc

N_NODES = 10000
NP = 10240
N_EDGES = 320000
IN_DIM = 128
HID_DIM = 64
NUM_REL = 2

NC = 2
NS = 16
NW = NC * NS

EDGE_B = 128
E_PAD = 327680
K_PER_W = E_PAD // (NW * EDGE_B)
NBUF = 2
ACC_ROWS = NUM_REL * NP
ROWS_PER_TILE = ACC_ROWS // NS
_EB = N_EDGES // 128
_EBP = E_PAD // 128

NB = 512
_NG = NP // NB


def _table_body(x_ref, wrel_ref, t_ref):
    t_ref[...] = jnp.dot(x_ref[...], wrel_ref[0],
                         preferred_element_type=jnp.float32)


def _make_table_call(in_dim):
    return pl.pallas_call(
        _table_body,
        grid=(NUM_REL, _NG),
        in_specs=[
            pl.BlockSpec((NB, in_dim), lambda r, i: (i, 0)),
            pl.BlockSpec((1, in_dim, HID_DIM), lambda r, i: (r, 0, 0)),
        ],
        out_specs=pl.BlockSpec((NB, HID_DIM), lambda r, i: (r * _NG + i, 0)),
        out_shape=jax.ShapeDtypeStruct((ACC_ROWS, HID_DIM), jnp.float32),
    )


_table1_call = _make_table_call(IN_DIM)
_table2_call = _make_table_call(HID_DIM)


def _root_body(x_ref, w_ref, b_ref, r_ref):
    r_ref[...] = (jnp.dot(x_ref[...], w_ref[...],
                          preferred_element_type=jnp.float32)
                  + b_ref[...][None, :])


def _make_root_call(in_dim):
    return pl.pallas_call(
        _root_body,
        grid=(_NG,),
        in_specs=[
            pl.BlockSpec((NB, in_dim), lambda i: (i, 0)),
            pl.BlockSpec((in_dim, HID_DIM), lambda i: (0, 0)),
            pl.BlockSpec((HID_DIM,), lambda i: (0,)),
        ],
        out_specs=pl.BlockSpec((NB, HID_DIM), lambda i: (i, 0)),
        out_shape=jax.ShapeDtypeStruct((NP, HID_DIM), jnp.float32),
    )


_root1_call = _make_root_call(IN_DIM)
_root2_call = _make_root_call(HID_DIM)


def _edge_body(ei_ref, et_ref, gidx_ref, sidx_ref):
    lanes = jax.lax.broadcasted_iota(jnp.int32, (_EBP - _EB, 128), 1)
    rows = jax.lax.broadcasted_iota(jnp.int32, (_EBP - _EB, 128), 0)
    pad = N_NODES + (rows * 128 + lanes) % (NP - N_NODES)
    et = et_ref[...]
    g = jnp.concatenate([et * NP + ei_ref[0], pad], axis=0)
    s = jnp.concatenate([et * NP + ei_ref[1], pad], axis=0)
    gidx_ref[...] = g.reshape(NW, K_PER_W, EDGE_B)
    sidx_ref[...] = s.reshape(NW, K_PER_W, EDGE_B)


_edge_call = pl.pallas_call(
    _edge_body,
    out_shape=(
        jax.ShapeDtypeStruct((NW, K_PER_W, EDGE_B), jnp.int32),
        jax.ShapeDtypeStruct((NW, K_PER_W, EDGE_B), jnp.int32),
    ),
)


def _layer_norm(h, g, b):
    mu = jnp.mean(h, axis=-1, keepdims=True)
    var = jnp.mean((h - mu) ** 2, axis=-1, keepdims=True)
    return (h - mu) / jnp.sqrt(var + 1e-5) * g + b


def _combine(acc_ref, cnt_ref):
    c0 = jnp.maximum(cnt_ref[0, 0, :, 0:1] + cnt_ref[1, 0, :, 0:1], 1.0)
    c1 = jnp.maximum(cnt_ref[0, 1, :, 0:1] + cnt_ref[1, 1, :, 0:1], 1.0)
    return ((acc_ref[0, 0] + acc_ref[1, 0]) / c0
            + (acc_ref[0, 1] + acc_ref[1, 1]) / c1)


_acc_spec = pl.BlockSpec((NC, NUM_REL, NB, HID_DIM), lambda *g: (0, 0, g[-1], 0))
_cnt_spec = pl.BlockSpec((NC, NUM_REL, NB, 8), lambda *g: (0, 0, g[-1], 0))
_vec_spec = pl.BlockSpec((HID_DIM,), lambda *g: (0,))
_row_spec = pl.BlockSpec((NB, HID_DIM), lambda *g: (g[-1], 0))


def _hid_body(r1_ref, acc_ref, cnt_ref, g_ref, bln_ref, h_ref):
    h = r1_ref[...] + _combine(acc_ref, cnt_ref)
    h = _layer_norm(h, g_ref[...][None, :], bln_ref[...][None, :])
    h_ref[...] = 0.5 * h * (1.0 + lax.erf(h * (2.0 ** -0.5)))


_hid_call = pl.pallas_call(
    _hid_body,
    grid=(_NG,),
    in_specs=[_row_spec, _acc_spec, _cnt_spec, _vec_spec, _vec_spec],
    out_specs=_row_spec,
    out_shape=jax.ShapeDtypeStruct((NP, HID_DIM), jnp.float32),
)

_FNB = 400
_facc_spec = pl.BlockSpec((NC, NUM_REL, _FNB, HID_DIM), lambda i: (0, 0, i, 0))
_fcnt_spec = pl.BlockSpec((NC, NUM_REL, _FNB, 8), lambda i: (0, 0, i, 0))
_frow_spec = pl.BlockSpec((_FNB, HID_DIM), lambda i: (i, 0))


def _final_body(r2_ref, acc_ref, cnt_ref, g_ref, bln_ref, out_ref):
    h = r2_ref[...] + _combine(acc_ref, cnt_ref)
    out_ref[...] = _layer_norm(h, g_ref[...][None, :], bln_ref[...][None, :])


_final_call = pl.pallas_call(
    _final_body,
    grid=(N_NODES // _FNB,),
    in_specs=[_frow_spec, _facc_spec, _fcnt_spec, _vec_spec, _vec_spec],
    out_specs=_frow_spec,
    out_shape=jax.ShapeDtypeStruct((N_NODES, HID_DIM), jnp.float32),
)


_SC_MESH = plsc.VectorSubcoreMesh(core_axis_name="c", subcore_axis_name="s")


def _sc_scatter_body(with_cnt, *refs):
    if with_cnt:
        (t_hbm, gidx_hbm, sidx_hbm, z64_hbm, z8_hbm, ones_hbm,
         acc_hbm, cnt_hbm, gidx_v, sidx_v, rows_v, ones_v, acc_sh, cnt_sh,
         *gsems) = refs
    else:
        (t_hbm, gidx_hbm, sidx_hbm, z64_hbm,
         acc_hbm, gidx_v, sidx_v, rows_v, acc_sh, *gsems) = refs
    c = lax.axis_index("c")
    s = lax.axis_index("s")
    w = c * NS + s

    pltpu.sync_copy(gidx_hbm.at[w], gidx_v)
    pltpu.sync_copy(sidx_hbm.at[w], sidx_v)
    for b in range(NBUF):
        pltpu.async_copy(t_hbm.at[gidx_v.at[b]], rows_v.at[b], gsems[b])
    pltpu.sync_copy(z64_hbm, acc_sh.at[pl.ds(s * ROWS_PER_TILE, ROWS_PER_TILE)])
    if with_cnt:
        pltpu.sync_copy(z8_hbm, cnt_sh.at[pl.ds(s * ROWS_PER_TILE, ROWS_PER_TILE)])
        pltpu.sync_copy(ones_hbm, ones_v)
    plsc.subcore_barrier()

    @pl.loop(0, K_PER_W, step=NBUF)
    def _grp(g):
        for b in range(NBUF):
            j = g + b
            pltpu.make_async_copy(t_hbm.at[gidx_v.at[j]], rows_v.at[b],
                                  gsems[b]).wait()
            pltpu.sync_copy(rows_v.at[b], acc_sh.at[sidx_v.at[j]], add=True)
            if with_cnt:
                pltpu.sync_copy(ones_v, cnt_sh.at[sidx_v.at[j]], add=True)

            @pl.when(j + NBUF < K_PER_W)
            def _refill():
                pltpu.async_copy(t_hbm.at[gidx_v.at[j + NBUF]], rows_v.at[b],
                                 gsems[b])

    plsc.subcore_barrier()
    r = s // 8
    o = (s % 8) * ROWS_PER_TILE
    sl = pl.ds(s * ROWS_PER_TILE, ROWS_PER_TILE)
    pltpu.sync_copy(acc_sh.at[sl], acc_hbm.at[c, r, pl.ds(o, ROWS_PER_TILE)])
    if with_cnt:
        pltpu.sync_copy(cnt_sh.at[sl], cnt_hbm.at[c, r, pl.ds(o, ROWS_PER_TILE)])


_sc_scatter_cnt = pl.kernel(
    functools.partial(_sc_scatter_body, True),
    out_type=(
        jax.ShapeDtypeStruct((NC, NUM_REL, NP, HID_DIM), jnp.float32),
        jax.ShapeDtypeStruct((NC, NUM_REL, NP, 8), jnp.float32),
    ),
    mesh=_SC_MESH,
    scratch_types=[
        pltpu.VMEM((K_PER_W, EDGE_B), jnp.int32),
        pltpu.VMEM((K_PER_W, EDGE_B), jnp.int32),
        pltpu.VMEM((NBUF, EDGE_B, HID_DIM), jnp.float32),
        pltpu.VMEM((EDGE_B, 8), jnp.float32),
        pltpu.VMEM_SHARED((ACC_ROWS, HID_DIM), jnp.float32),
        pltpu.VMEM_SHARED((ACC_ROWS, 8), jnp.float32),
    ] + [pltpu.SemaphoreType.DMA] * NBUF,
    compiler_params=pltpu.CompilerParams(use_tc_tiling_on_sc=False),
)

_sc_scatter_nocnt = pl.kernel(
    functools.partial(_sc_scatter_body, False),
    out_type=jax.ShapeDtypeStruct((NC, NUM_REL, NP, HID_DIM), jnp.float32),
    mesh=_SC_MESH,
    scratch_types=[
        pltpu.VMEM((K_PER_W, EDGE_B), jnp.int32),
        pltpu.VMEM((K_PER_W, EDGE_B), jnp.int32),
        pltpu.VMEM((NBUF, EDGE_B, HID_DIM), jnp.float32),
        pltpu.VMEM_SHARED((ACC_ROWS, HID_DIM), jnp.float32),
    ] + [pltpu.SemaphoreType.DMA] * NBUF,
    compiler_params=pltpu.CompilerParams(use_tc_tiling_on_sc=False),
)


def kernel(x, edge_index, edge_type, W_rel1, W_root1, b1, ln1_g, ln1_b,
           W_rel2, W_root2, b2, ln2_g, ln2_b):
    xp = jnp.pad(x, ((0, NP - N_NODES), (0, 0)))
    ei = edge_index.reshape(2, _EB, 128)
    et = edge_type.reshape(_EB, 128)

    t1 = _table1_call(xp, W_rel1)
    gidx, sidx = _edge_call(ei, et)

    z64 = jnp.zeros((ROWS_PER_TILE, HID_DIM), jnp.float32)
    z8 = jnp.zeros((ROWS_PER_TILE, 8), jnp.float32)
    ones8 = jnp.ones((EDGE_B, 8), jnp.float32)

    acc1, cnt = _sc_scatter_cnt(t1, gidx, sidx, z64, z8, ones8)
    r1 = _root1_call(xp, W_root1, b1)
    h = _hid_call(r1, acc1, cnt, ln1_g, ln1_b)
    t2 = _table2_call(h, W_rel2)
    acc2 = _sc_scatter_nocnt(t2, gidx, sidx, z64)
    r2 = _root2_call(h, W_root2, b2)
    return _final_call(r2, acc2, cnt, ln2_g, ln2_b)


# speedup vs baseline: 1.3495x; 1.3495x over previous
import functools

import jax
import jax.numpy as jnp
from jax import lax
from jax.experimental import pallas as pl
from jax.experimental.pallas import tpu as pltpu
from jax.experimental.pallas import tpu_sc as plsc

N_NODES = 10000
NP = 10240
NPH = NP // 2
N_EDGES = 320000
IN_DIM = 128
HID_DIM = 64
NUM_REL = 2

NC = 2
NS = 16
NW = NC * NS

EDGE_B = 128
E_PAD = 327680
K_PER_W = E_PAD // (NW * EDGE_B)
NBUF = 2
ACC_ROWS = NUM_REL * NP
ROWS_PER_TILE = ACC_ROWS // NS
_EB = N_EDGES // 128
_EBP = E_PAD // 128

NBH = 512
_NG = NPH // NBH


def _table_body(x_ref, wbd_ref, t_ref):
    t_ref[...] = jnp.dot(x_ref[...], wbd_ref[0],
                         preferred_element_type=jnp.float32)


def _make_table_call(in_pair_dim):
    return pl.pallas_call(
        _table_body,
        grid=(NUM_REL, _NG),
        in_specs=[
            pl.BlockSpec((NBH, in_pair_dim), lambda r, i: (i, 0)),
            pl.BlockSpec((1, in_pair_dim, 128), lambda r, i: (r, 0, 0)),
        ],
        out_specs=pl.BlockSpec((NBH, 128), lambda r, i: (r * _NG + i, 0)),
        out_shape=jax.ShapeDtypeStruct((ACC_ROWS // 2, 128), jnp.float32),
    )


_table1_call = _make_table_call(2 * IN_DIM)
_table2_call = _make_table_call(2 * HID_DIM)


def _root_body(x_ref, wbd_ref, b_ref, r_ref):
    r_ref[...] = (jnp.dot(x_ref[...], wbd_ref[...],
                          preferred_element_type=jnp.float32)
                  + b_ref[...][None, :])


def _make_root_call(in_pair_dim):
    return pl.pallas_call(
        _root_body,
        grid=(_NG,),
        in_specs=[
            pl.BlockSpec((NBH, in_pair_dim), lambda i: (i, 0)),
            pl.BlockSpec((in_pair_dim, 128), lambda i: (0, 0)),
            pl.BlockSpec((128,), lambda i: (0,)),
        ],
        out_specs=pl.BlockSpec((NBH, 128), lambda i: (i, 0)),
        out_shape=jax.ShapeDtypeStruct((NPH, 128), jnp.float32),
    )


_root1_call = _make_root_call(2 * IN_DIM)
_root2_call = _make_root_call(2 * HID_DIM)


def _edge_body(ei_ref, et_ref, gidx_ref, sidx_ref):
    lanes = jax.lax.broadcasted_iota(jnp.int32, (_EBP - _EB, 128), 1)
    rows = jax.lax.broadcasted_iota(jnp.int32, (_EBP - _EB, 128), 0)
    pad = N_NODES + (rows * 128 + lanes) % (NP - N_NODES)
    et = et_ref[...]
    g = jnp.concatenate([et * NP + ei_ref[0], pad], axis=0)
    s = jnp.concatenate([et * NP + ei_ref[1], pad], axis=0)
    gidx_ref[...] = g.reshape(NW, K_PER_W, EDGE_B)
    sidx_ref[...] = s.reshape(NW, K_PER_W, EDGE_B)


_edge_call = pl.pallas_call(
    _edge_body,
    out_shape=(
        jax.ShapeDtypeStruct((NW, K_PER_W, EDGE_B), jnp.int32),
        jax.ShapeDtypeStruct((NW, K_PER_W, EDGE_B), jnp.int32),
    ),
)


def _pair_divisor(cnt_ref, r, n):
    c_even = jnp.maximum(cnt_ref[0, r, :, 0:1] + cnt_ref[1, r, :, 0:1], 1.0)
    c_odd = jnp.maximum(cnt_ref[0, r, :, 8:9] + cnt_ref[1, r, :, 8:9], 1.0)
    return jnp.concatenate([jnp.broadcast_to(c_even, (n, HID_DIM)),
                            jnp.broadcast_to(c_odd, (n, HID_DIM))], axis=1)


def _combine(acc_ref, cnt_ref, n):
    return ((acc_ref[0, 0] + acc_ref[1, 0]) / _pair_divisor(cnt_ref, 0, n)
            + (acc_ref[0, 1] + acc_ref[1, 1]) / _pair_divisor(cnt_ref, 1, n))


def _ln_half(h, g, b):
    mu = jnp.mean(h, axis=-1, keepdims=True)
    var = jnp.mean((h - mu) ** 2, axis=-1, keepdims=True)
    return (h - mu) / jnp.sqrt(var + 1e-5) * g + b


def _layer_norm_pair(h, g, b):
    return jnp.concatenate([_ln_half(h[:, 0:HID_DIM], g, b),
                            _ln_half(h[:, HID_DIM:128], g, b)], axis=1)


def _hid_body(r1_ref, acc_ref, cnt_ref, g_ref, bln_ref, h_ref):
    h = r1_ref[...] + _combine(acc_ref, cnt_ref, NBH)
    h = _layer_norm_pair(h, g_ref[...][None, :], bln_ref[...][None, :])
    h_ref[...] = 0.5 * h * (1.0 + lax.erf(h * (2.0 ** -0.5)))


_acc_spec = pl.BlockSpec((NC, NUM_REL, NBH, 128), lambda *g: (0, 0, g[-1], 0))
_cnt_spec = pl.BlockSpec((NC, NUM_REL, NBH, 16), lambda *g: (0, 0, g[-1], 0))
_vec_spec = pl.BlockSpec((HID_DIM,), lambda *g: (0,))
_row_spec = pl.BlockSpec((NBH, 128), lambda *g: (g[-1], 0))

_hid_call = pl.pallas_call(
    _hid_body,
    grid=(_NG,),
    in_specs=[_row_spec, _acc_spec, _cnt_spec, _vec_spec, _vec_spec],
    out_specs=_row_spec,
    out_shape=jax.ShapeDtypeStruct((NPH, 128), jnp.float32),
)

_FNB = 200


def _final_body(r2_ref, acc_ref, cnt_ref, g_ref, bln_ref, out_ref):
    h = r2_ref[...] + _combine(acc_ref, cnt_ref, _FNB)
    out_ref[...] = _layer_norm_pair(h, g_ref[...][None, :], bln_ref[...][None, :])


_final_call = pl.pallas_call(
    _final_body,
    grid=(N_NODES // (2 * _FNB),),
    in_specs=[
        pl.BlockSpec((_FNB, 128), lambda i: (i, 0)),
        pl.BlockSpec((NC, NUM_REL, _FNB, 128), lambda i: (0, 0, i, 0)),
        pl.BlockSpec((NC, NUM_REL, _FNB, 16), lambda i: (0, 0, i, 0)),
        _vec_spec,
        _vec_spec,
    ],
    out_specs=pl.BlockSpec((_FNB, 128), lambda i: (i, 0)),
    out_shape=jax.ShapeDtypeStruct((N_NODES // 2, 128), jnp.float32),
)


_SC_MESH = plsc.VectorSubcoreMesh(core_axis_name="c", subcore_axis_name="s")


def _sc_scatter_body(with_cnt, *refs):
    if with_cnt:
        (t_hbm, gidx_hbm, sidx_hbm, z64_hbm, z8_hbm, ones_hbm,
         acc_hbm, cnt_hbm, gidx_v, sidx_v, rows_v, ones_v, acc_sh, cnt_sh,
         *gsems) = refs
    else:
        (t_hbm, gidx_hbm, sidx_hbm, z64_hbm,
         acc_hbm, gidx_v, sidx_v, rows_v, acc_sh, *gsems) = refs
    c = lax.axis_index("c")
    s = lax.axis_index("s")
    w = c * NS + s

    pltpu.sync_copy(gidx_hbm.at[w], gidx_v)
    pltpu.sync_copy(sidx_hbm.at[w], sidx_v)
    for b in range(NBUF):
        pltpu.async_copy(t_hbm.at[gidx_v.at[b]], rows_v.at[b], gsems[b])
    pltpu.sync_copy(z64_hbm, acc_sh.at[pl.ds(s * ROWS_PER_TILE, ROWS_PER_TILE)])
    if with_cnt:
        pltpu.sync_copy(z8_hbm, cnt_sh.at[pl.ds(s * ROWS_PER_TILE, ROWS_PER_TILE)])
        pltpu.sync_copy(ones_hbm, ones_v)
    plsc.subcore_barrier()

    @pl.loop(0, K_PER_W, step=NBUF)
    def _grp(g):
        for b in range(NBUF):
            j = g + b
            pltpu.make_async_copy(t_hbm.at[gidx_v.at[j]], rows_v.at[b],
                                  gsems[b]).wait()
            pltpu.sync_copy(rows_v.at[b], acc_sh.at[sidx_v.at[j]], add=True)
            if with_cnt:
                pltpu.sync_copy(ones_v, cnt_sh.at[sidx_v.at[j]], add=True)

            @pl.when(j + NBUF < K_PER_W)
            def _refill():
                pltpu.async_copy(t_hbm.at[gidx_v.at[j + NBUF]], rows_v.at[b],
                                 gsems[b])

    plsc.subcore_barrier()
    r = s // 8
    o = (s % 8) * ROWS_PER_TILE
    sl = pl.ds(s * ROWS_PER_TILE, ROWS_PER_TILE)
    pltpu.sync_copy(acc_sh.at[sl], acc_hbm.at[c, r, pl.ds(o, ROWS_PER_TILE)])
    if with_cnt:
        pltpu.sync_copy(cnt_sh.at[sl], cnt_hbm.at[c, r, pl.ds(o, ROWS_PER_TILE)])


_sc_scatter_cnt = pl.kernel(
    functools.partial(_sc_scatter_body, True),
    out_type=(
        jax.ShapeDtypeStruct((NC, NUM_REL, NP, HID_DIM), jnp.float32),
        jax.ShapeDtypeStruct((NC, NUM_REL, NP, 8), jnp.float32),
    ),
    mesh=_SC_MESH,
    scratch_types=[
        pltpu.VMEM((K_PER_W, EDGE_B), jnp.int32),
        pltpu.VMEM((K_PER_W, EDGE_B), jnp.int32),
        pltpu.VMEM((NBUF, EDGE_B, HID_DIM), jnp.float32),
        pltpu.VMEM((EDGE_B, 8), jnp.float32),
        pltpu.VMEM_SHARED((ACC_ROWS, HID_DIM), jnp.float32),
        pltpu.VMEM_SHARED((ACC_ROWS, 8), jnp.float32),
    ] + [pltpu.SemaphoreType.DMA] * NBUF,
    compiler_params=pltpu.CompilerParams(use_tc_tiling_on_sc=False),
)

_sc_scatter_nocnt = pl.kernel(
    functools.partial(_sc_scatter_body, False),
    out_type=jax.ShapeDtypeStruct((NC, NUM_REL, NP, HID_DIM), jnp.float32),
    mesh=_SC_MESH,
    scratch_types=[
        pltpu.VMEM((K_PER_W, EDGE_B), jnp.int32),
        pltpu.VMEM((K_PER_W, EDGE_B), jnp.int32),
        pltpu.VMEM((NBUF, EDGE_B, HID_DIM), jnp.float32),
        pltpu.VMEM_SHARED((ACC_ROWS, HID_DIM), jnp.float32),
    ] + [pltpu.SemaphoreType.DMA] * NBUF,
    compiler_params=pltpu.CompilerParams(use_tc_tiling_on_sc=False),
)


def _blockdiag(w):
    z = jnp.zeros_like(w)
    return jnp.concatenate([jnp.concatenate([w, z], axis=1),
                            jnp.concatenate([z, w], axis=1)], axis=0)


def kernel(x, edge_index, edge_type, W_rel1, W_root1, b1, ln1_g, ln1_b,
           W_rel2, W_root2, b2, ln2_g, ln2_b):
    x_pair = jnp.pad(x, ((0, NP - N_NODES), (0, 0))).reshape(NPH, 2 * IN_DIM)
    ei = edge_index.reshape(2, _EB, 128)
    et = edge_type.reshape(_EB, 128)

    wrel1_bd = jnp.stack([_blockdiag(W_rel1[0]), _blockdiag(W_rel1[1])])
    wrel2_bd = jnp.stack([_blockdiag(W_rel2[0]), _blockdiag(W_rel2[1])])
    wroot1_bd = _blockdiag(W_root1)
    wroot2_bd = _blockdiag(W_root2)
    b1p = jnp.concatenate([b1, b1])
    b2p = jnp.concatenate([b2, b2])

    t1 = _table1_call(x_pair, wrel1_bd).reshape(ACC_ROWS, HID_DIM)
    gidx, sidx = _edge_call(ei, et)

    z64 = jnp.zeros((ROWS_PER_TILE, HID_DIM), jnp.float32)
    z8 = jnp.zeros((ROWS_PER_TILE, 8), jnp.float32)
    ones8 = jnp.ones((EDGE_B, 8), jnp.float32)

    acc1, cnt = _sc_scatter_cnt(t1, gidx, sidx, z64, z8, ones8)
    acc1p = acc1.reshape(NC, NUM_REL, NPH, 128)
    cntp = cnt.reshape(NC, NUM_REL, NPH, 16)
    r1 = _root1_call(x_pair, wroot1_bd, b1p)

    h = _hid_call(r1, acc1p, cntp, ln1_g, ln1_b)
    t2 = _table2_call(h, wrel2_bd).reshape(ACC_ROWS, HID_DIM)

    acc2 = _sc_scatter_nocnt(t2, gidx, sidx, z64)
    acc2p = acc2.reshape(NC, NUM_REL, NPH, 128)
    r2 = _root2_call(h, wroot2_bd, b2p)

    out = _final_call(r2, acc2p, cntp, ln2_g, ln2_b)
    return out.reshape(N_NODES, HID_DIM)


# speedup vs baseline: 1.4696x; 1.0890x over previous
import functools

import jax
import jax.numpy as jnp
from jax import lax
from jax.experimental import pallas as pl
from jax.experimental.pallas import tpu as pltpu
from jax.experimental.pallas import tpu_sc as plsc

N_NODES = 10000
NP = 10240
NPH = NP // 2
N_EDGES = 320000
IN_DIM = 128
HID_DIM = 64
NUM_REL = 2

NC = 2
NS = 16
NW = NC * NS

EDGE_B = 128
E_PAD = 327680
K_PER_W = E_PAD // (NW * EDGE_B)
NBUF = 2
ACC_ROWS = NUM_REL * NP
ROWS_PER_TILE = ACC_ROWS // NS
_EB = N_EDGES // 128
_EBP = E_PAD // 128

NBH = 1024
_NG = NPH // NBH


def _table_body(x_ref, wbd_ref, t_ref):
    t_ref[...] = jnp.dot(x_ref[...], wbd_ref[0],
                         preferred_element_type=jnp.float32)


def _make_table_call(in_pair_dim):
    return pl.pallas_call(
        _table_body,
        grid=(NUM_REL, _NG),
        in_specs=[
            pl.BlockSpec((NBH, in_pair_dim), lambda r, i: (i, 0)),
            pl.BlockSpec((1, in_pair_dim, 128), lambda r, i: (r, 0, 0)),
        ],
        out_specs=pl.BlockSpec((NBH, 128), lambda r, i: (r * _NG + i, 0)),
        out_shape=jax.ShapeDtypeStruct((ACC_ROWS // 2, 128), jnp.float32),
    )


_table1_call = _make_table_call(2 * IN_DIM)
_table2_call = _make_table_call(2 * HID_DIM)


def _root_body(x_ref, wbd_ref, b_ref, r_ref):
    r_ref[...] = (jnp.dot(x_ref[...], wbd_ref[...],
                          preferred_element_type=jnp.float32)
                  + b_ref[...][None, :])


def _make_root_call(in_pair_dim):
    return pl.pallas_call(
        _root_body,
        grid=(_NG,),
        in_specs=[
            pl.BlockSpec((NBH, in_pair_dim), lambda i: (i, 0)),
            pl.BlockSpec((in_pair_dim, 128), lambda i: (0, 0)),
            pl.BlockSpec((128,), lambda i: (0,)),
        ],
        out_specs=pl.BlockSpec((NBH, 128), lambda i: (i, 0)),
        out_shape=jax.ShapeDtypeStruct((NPH, 128), jnp.float32),
    )


_root1_call = _make_root_call(2 * IN_DIM)
_root2_call = _make_root_call(2 * HID_DIM)


def _edge_body(ei_ref, et_ref, gidx_ref, sidx_ref):
    lanes = jax.lax.broadcasted_iota(jnp.int32, (_EBP - _EB, 128), 1)
    rows = jax.lax.broadcasted_iota(jnp.int32, (_EBP - _EB, 128), 0)
    pad = N_NODES + (rows * 128 + lanes) % (NP - N_NODES)
    et = et_ref[...]
    g = jnp.concatenate([et * NP + ei_ref[0], pad], axis=0)
    s = jnp.concatenate([et * NP + ei_ref[1], pad], axis=0)
    gidx_ref[...] = g.reshape(NW, K_PER_W, EDGE_B)
    sidx_ref[...] = s.reshape(NW, K_PER_W, EDGE_B)


_edge_call = pl.pallas_call(
    _edge_body,
    out_shape=(
        jax.ShapeDtypeStruct((NW, K_PER_W, EDGE_B), jnp.int32),
        jax.ShapeDtypeStruct((NW, K_PER_W, EDGE_B), jnp.int32),
    ),
)


def _pair_divisor(cnt_ref, r, n):
    c_even = jnp.maximum(cnt_ref[0, r, :, 0:1] + cnt_ref[1, r, :, 0:1], 1.0)
    c_odd = jnp.maximum(cnt_ref[0, r, :, 8:9] + cnt_ref[1, r, :, 8:9], 1.0)
    return jnp.concatenate([jnp.broadcast_to(c_even, (n, HID_DIM)),
                            jnp.broadcast_to(c_odd, (n, HID_DIM))], axis=1)


def _combine(acc_ref, cnt_ref, n):
    return ((acc_ref[0, 0] + acc_ref[1, 0]) / _pair_divisor(cnt_ref, 0, n)
            + (acc_ref[0, 1] + acc_ref[1, 1]) / _pair_divisor(cnt_ref, 1, n))


def _ln_half(h, g, b):
    mu = jnp.mean(h, axis=-1, keepdims=True)
    var = jnp.mean((h - mu) ** 2, axis=-1, keepdims=True)
    return (h - mu) / jnp.sqrt(var + 1e-5) * g + b


def _layer_norm_pair(h, g, b):
    return jnp.concatenate([_ln_half(h[:, 0:HID_DIM], g, b),
                            _ln_half(h[:, HID_DIM:128], g, b)], axis=1)


def _hid_body(r1_ref, acc_ref, cnt_ref, g_ref, bln_ref, h_ref):
    h = r1_ref[...] + _combine(acc_ref, cnt_ref, NBH)
    h = _layer_norm_pair(h, g_ref[...][None, :], bln_ref[...][None, :])
    h_ref[...] = 0.5 * h * (1.0 + lax.erf(h * (2.0 ** -0.5)))


_acc_spec = pl.BlockSpec((NC, NUM_REL, NBH, 128), lambda *g: (0, 0, g[-1], 0))
_cnt_spec = pl.BlockSpec((NC, NUM_REL, NBH, 16), lambda *g: (0, 0, g[-1], 0))
_vec_spec = pl.BlockSpec((HID_DIM,), lambda *g: (0,))
_row_spec = pl.BlockSpec((NBH, 128), lambda *g: (g[-1], 0))

_hid_call = pl.pallas_call(
    _hid_body,
    grid=(_NG,),
    in_specs=[_row_spec, _acc_spec, _cnt_spec, _vec_spec, _vec_spec],
    out_specs=_row_spec,
    out_shape=jax.ShapeDtypeStruct((NPH, 128), jnp.float32),
)

_FNB = 1000


def _final_body(r2_ref, acc_ref, cnt_ref, g_ref, bln_ref, out_ref):
    h = r2_ref[...] + _combine(acc_ref, cnt_ref, _FNB)
    out_ref[...] = _layer_norm_pair(h, g_ref[...][None, :], bln_ref[...][None, :])


_final_call = pl.pallas_call(
    _final_body,
    grid=(N_NODES // (2 * _FNB),),
    in_specs=[
        pl.BlockSpec((_FNB, 128), lambda i: (i, 0)),
        pl.BlockSpec((NC, NUM_REL, _FNB, 128), lambda i: (0, 0, i, 0)),
        pl.BlockSpec((NC, NUM_REL, _FNB, 16), lambda i: (0, 0, i, 0)),
        _vec_spec,
        _vec_spec,
    ],
    out_specs=pl.BlockSpec((_FNB, 128), lambda i: (i, 0)),
    out_shape=jax.ShapeDtypeStruct((N_NODES // 2, 128), jnp.float32),
)


_SC_MESH = plsc.VectorSubcoreMesh(core_axis_name="c", subcore_axis_name="s")


def _sc_scatter_body(with_cnt, *refs):
    if with_cnt:
        (t_hbm, gidx_hbm, sidx_hbm, z64_hbm, z8_hbm, ones_hbm,
         acc_hbm, cnt_hbm, gidx_v, sidx_v, rows_v, ones_v, acc_sh, cnt_sh,
         *gsems) = refs
    else:
        (t_hbm, gidx_hbm, sidx_hbm, z64_hbm,
         acc_hbm, gidx_v, sidx_v, rows_v, acc_sh, *gsems) = refs
    c = lax.axis_index("c")
    s = lax.axis_index("s")
    w = c * NS + s

    pltpu.sync_copy(gidx_hbm.at[w], gidx_v)
    pltpu.sync_copy(sidx_hbm.at[w], sidx_v)
    for b in range(NBUF):
        pltpu.async_copy(t_hbm.at[gidx_v.at[b]], rows_v.at[b], gsems[b])
    pltpu.sync_copy(z64_hbm, acc_sh.at[pl.ds(s * ROWS_PER_TILE, ROWS_PER_TILE)])
    if with_cnt:
        pltpu.sync_copy(z8_hbm, cnt_sh.at[pl.ds(s * ROWS_PER_TILE, ROWS_PER_TILE)])
        pltpu.sync_copy(ones_hbm, ones_v)
    plsc.subcore_barrier()

    @pl.loop(0, K_PER_W, step=NBUF)
    def _grp(g):
        for b in range(NBUF):
            j = g + b
            pltpu.make_async_copy(t_hbm.at[gidx_v.at[j]], rows_v.at[b],
                                  gsems[b]).wait()
            pltpu.sync_copy(rows_v.at[b], acc_sh.at[sidx_v.at[j]], add=True)
            if with_cnt:
                pltpu.sync_copy(ones_v, cnt_sh.at[sidx_v.at[j]], add=True)

            @pl.when(j + NBUF < K_PER_W)
            def _refill():
                pltpu.async_copy(t_hbm.at[gidx_v.at[j + NBUF]], rows_v.at[b],
                                 gsems[b])

    plsc.subcore_barrier()
    r = s // 8
    o = (s % 8) * ROWS_PER_TILE
    sl = pl.ds(s * ROWS_PER_TILE, ROWS_PER_TILE)
    pltpu.sync_copy(acc_sh.at[sl], acc_hbm.at[c, r, pl.ds(o, ROWS_PER_TILE)])
    if with_cnt:
        pltpu.sync_copy(cnt_sh.at[sl], cnt_hbm.at[c, r, pl.ds(o, ROWS_PER_TILE)])


_sc_scatter_cnt = pl.kernel(
    functools.partial(_sc_scatter_body, True),
    out_type=(
        jax.ShapeDtypeStruct((NC, NUM_REL, NP, HID_DIM), jnp.float32),
        jax.ShapeDtypeStruct((NC, NUM_REL, NP, 8), jnp.float32),
    ),
    mesh=_SC_MESH,
    scratch_types=[
        pltpu.VMEM((K_PER_W, EDGE_B), jnp.int32),
        pltpu.VMEM((K_PER_W, EDGE_B), jnp.int32),
        pltpu.VMEM((NBUF, EDGE_B, HID_DIM), jnp.float32),
        pltpu.VMEM((EDGE_B, 8), jnp.float32),
        pltpu.VMEM_SHARED((ACC_ROWS, HID_DIM), jnp.float32),
        pltpu.VMEM_SHARED((ACC_ROWS, 8), jnp.float32),
    ] + [pltpu.SemaphoreType.DMA] * NBUF,
    compiler_params=pltpu.CompilerParams(use_tc_tiling_on_sc=False),
)

_sc_scatter_nocnt = pl.kernel(
    functools.partial(_sc_scatter_body, False),
    out_type=jax.ShapeDtypeStruct((NC, NUM_REL, NP, HID_DIM), jnp.float32),
    mesh=_SC_MESH,
    scratch_types=[
        pltpu.VMEM((K_PER_W, EDGE_B), jnp.int32),
        pltpu.VMEM((K_PER_W, EDGE_B), jnp.int32),
        pltpu.VMEM((NBUF, EDGE_B, HID_DIM), jnp.float32),
        pltpu.VMEM_SHARED((ACC_ROWS, HID_DIM), jnp.float32),
    ] + [pltpu.SemaphoreType.DMA] * NBUF,
    compiler_params=pltpu.CompilerParams(use_tc_tiling_on_sc=False),
)


def _blockdiag(w):
    z = jnp.zeros_like(w)
    return jnp.concatenate([jnp.concatenate([w, z], axis=1),
                            jnp.concatenate([z, w], axis=1)], axis=0)


def kernel(x, edge_index, edge_type, W_rel1, W_root1, b1, ln1_g, ln1_b,
           W_rel2, W_root2, b2, ln2_g, ln2_b):
    x_pair = jnp.pad(x, ((0, NP - N_NODES), (0, 0))).reshape(NPH, 2 * IN_DIM)
    ei = edge_index.reshape(2, _EB, 128)
    et = edge_type.reshape(_EB, 128)

    wrel1_bd = jnp.stack([_blockdiag(W_rel1[0]), _blockdiag(W_rel1[1])])
    wrel2_bd = jnp.stack([_blockdiag(W_rel2[0]), _blockdiag(W_rel2[1])])
    wroot1_bd = _blockdiag(W_root1)
    wroot2_bd = _blockdiag(W_root2)
    b1p = jnp.concatenate([b1, b1])
    b2p = jnp.concatenate([b2, b2])

    t1 = _table1_call(x_pair, wrel1_bd).reshape(ACC_ROWS, HID_DIM)
    gidx, sidx = _edge_call(ei, et)

    z64 = jnp.zeros((ROWS_PER_TILE, HID_DIM), jnp.float32)
    z8 = jnp.zeros((ROWS_PER_TILE, 8), jnp.float32)
    ones8 = jnp.ones((EDGE_B, 8), jnp.float32)

    acc1, cnt = _sc_scatter_cnt(t1, gidx, sidx, z64, z8, ones8)
    acc1p = acc1.reshape(NC, NUM_REL, NPH, 128)
    cntp = cnt.reshape(NC, NUM_REL, NPH, 16)
    r1 = _root1_call(x_pair, wroot1_bd, b1p)

    h = _hid_call(r1, acc1p, cntp, ln1_g, ln1_b)
    t2 = _table2_call(h, wrel2_bd).reshape(ACC_ROWS, HID_DIM)

    acc2 = _sc_scatter_nocnt(t2, gidx, sidx, z64)
    acc2p = acc2.reshape(NC, NUM_REL, NPH, 128)
    r2 = _root2_call(h, wroot2_bd, b2p)

    out = _final_call(r2, acc2p, cntp, ln2_g, ln2_b)
    return out.reshape(N_NODES, HID_DIM)


# speedup vs baseline: 1.5011x; 1.0214x over previous
import functools

import jax
import jax.numpy as jnp
from jax import lax
from jax.experimental import pallas as pl
from jax.experimental.pallas import tpu as pltpu
from jax.experimental.pallas import tpu_sc as plsc

N_NODES = 10000
NP = 10240
NPH = NP // 2
N_EDGES = 320000
IN_DIM = 128
HID_DIM = 64
NUM_REL = 2

NC = 2
NS = 16
NW = NC * NS

EDGE_B = 128
E_PAD = 327680
K_PER_W = E_PAD // (NW * EDGE_B)
NBUF = 2
ACC_ROWS = NUM_REL * NP
ROWS_PER_TILE = ACC_ROWS // NS
_EB = N_EDGES // 128
_EBP = E_PAD // 128

NBH = 1024
_NG = NPH // NBH


def _table_body(x_ref, wbd_ref, t_ref):
    t_ref[...] = jnp.dot(x_ref[...], wbd_ref[0],
                         preferred_element_type=jnp.float32)


def _make_table_call(in_pair_dim):
    return pl.pallas_call(
        _table_body,
        grid=(NUM_REL, _NG),
        in_specs=[
            pl.BlockSpec((NBH, in_pair_dim), lambda r, i: (i, 0)),
            pl.BlockSpec((1, in_pair_dim, 128), lambda r, i: (r, 0, 0)),
        ],
        out_specs=pl.BlockSpec((NBH, 128), lambda r, i: (r * _NG + i, 0)),
        out_shape=jax.ShapeDtypeStruct((ACC_ROWS // 2, 128), jnp.float32),
    )


_table1_call = _make_table_call(2 * IN_DIM)
_table2_call = _make_table_call(2 * HID_DIM)


def _root_body(x_ref, wbd_ref, b_ref, r_ref):
    r_ref[...] = (jnp.dot(x_ref[...], wbd_ref[...],
                          preferred_element_type=jnp.float32)
                  + b_ref[...][None, :])


def _make_root_call(in_pair_dim):
    return pl.pallas_call(
        _root_body,
        grid=(_NG,),
        in_specs=[
            pl.BlockSpec((NBH, in_pair_dim), lambda i: (i, 0)),
            pl.BlockSpec((in_pair_dim, 128), lambda i: (0, 0)),
            pl.BlockSpec((128,), lambda i: (0,)),
        ],
        out_specs=pl.BlockSpec((NBH, 128), lambda i: (i, 0)),
        out_shape=jax.ShapeDtypeStruct((NPH, 128), jnp.float32),
    )


_root1_call = _make_root_call(2 * IN_DIM)
_root2_call = _make_root_call(2 * HID_DIM)


def _edge_body(ei_ref, et_ref, gidx_ref, sidx_ref):
    lanes = jax.lax.broadcasted_iota(jnp.int32, (_EBP - _EB, 128), 1)
    rows = jax.lax.broadcasted_iota(jnp.int32, (_EBP - _EB, 128), 0)
    pad = N_NODES + (rows * 128 + lanes) % (NP - N_NODES)
    et = et_ref[...].reshape(_EB, 128)
    src = ei_ref[0].reshape(_EB, 128)
    dst = ei_ref[1].reshape(_EB, 128)
    g = jnp.concatenate([et * NP + src, pad], axis=0)
    s = jnp.concatenate([et * NP + dst, pad], axis=0)
    gidx_ref[...] = g.reshape(NW, K_PER_W, EDGE_B)
    sidx_ref[...] = s.reshape(NW, K_PER_W, EDGE_B)


_edge_call = pl.pallas_call(
    _edge_body,
    out_shape=(
        jax.ShapeDtypeStruct((NW, K_PER_W, EDGE_B), jnp.int32),
        jax.ShapeDtypeStruct((NW, K_PER_W, EDGE_B), jnp.int32),
    ),
)


def _pair_divisor(cnt_ref, r, n):
    c_even = jnp.maximum(cnt_ref[0, r, :, 0:1] + cnt_ref[1, r, :, 0:1], 1.0)
    c_odd = jnp.maximum(cnt_ref[0, r, :, 8:9] + cnt_ref[1, r, :, 8:9], 1.0)
    return jnp.concatenate([jnp.broadcast_to(c_even, (n, HID_DIM)),
                            jnp.broadcast_to(c_odd, (n, HID_DIM))], axis=1)


def _combine(acc_ref, cnt_ref, n):
    return ((acc_ref[0, 0] + acc_ref[1, 0]) / _pair_divisor(cnt_ref, 0, n)
            + (acc_ref[0, 1] + acc_ref[1, 1]) / _pair_divisor(cnt_ref, 1, n))


def _ln_half(h, g, b):
    mu = jnp.mean(h, axis=-1, keepdims=True)
    var = jnp.mean((h - mu) ** 2, axis=-1, keepdims=True)
    return (h - mu) / jnp.sqrt(var + 1e-5) * g + b


def _layer_norm_pair(h, g, b):
    return jnp.concatenate([_ln_half(h[:, 0:HID_DIM], g, b),
                            _ln_half(h[:, HID_DIM:128], g, b)], axis=1)


def _hid_body(r1_ref, acc_ref, cnt_ref, g_ref, bln_ref, h_ref):
    h = r1_ref[...] + _combine(acc_ref, cnt_ref, NBH)
    h = _layer_norm_pair(h, g_ref[...][None, :], bln_ref[...][None, :])
    h_ref[...] = 0.5 * h * (1.0 + lax.erf(h * (2.0 ** -0.5)))


_acc_spec = pl.BlockSpec((NC, NUM_REL, NBH, 128), lambda *g: (0, 0, g[-1], 0))
_cnt_spec = pl.BlockSpec((NC, NUM_REL, NBH, 16), lambda *g: (0, 0, g[-1], 0))
_vec_spec = pl.BlockSpec((HID_DIM,), lambda *g: (0,))
_row_spec = pl.BlockSpec((NBH, 128), lambda *g: (g[-1], 0))

_hid_call = pl.pallas_call(
    _hid_body,
    grid=(_NG,),
    in_specs=[_row_spec, _acc_spec, _cnt_spec, _vec_spec, _vec_spec],
    out_specs=_row_spec,
    out_shape=jax.ShapeDtypeStruct((NPH, 128), jnp.float32),
)

_FNB = 1000


def _final_body(r2_ref, acc_ref, cnt_ref, g_ref, bln_ref, out_ref):
    h = r2_ref[...] + _combine(acc_ref, cnt_ref, _FNB)
    out_ref[...] = _layer_norm_pair(h, g_ref[...][None, :], bln_ref[...][None, :])


_final_call = pl.pallas_call(
    _final_body,
    grid=(N_NODES // (2 * _FNB),),
    in_specs=[
        pl.BlockSpec((_FNB, 128), lambda i: (i, 0)),
        pl.BlockSpec((NC, NUM_REL, _FNB, 128), lambda i: (0, 0, i, 0)),
        pl.BlockSpec((NC, NUM_REL, _FNB, 16), lambda i: (0, 0, i, 0)),
        _vec_spec,
        _vec_spec,
    ],
    out_specs=pl.BlockSpec((_FNB, 128), lambda i: (i, 0)),
    out_shape=jax.ShapeDtypeStruct((N_NODES // 2, 128), jnp.float32),
)


_SC_MESH = plsc.VectorSubcoreMesh(core_axis_name="c", subcore_axis_name="s")


def _sc_scatter_body(with_cnt, *refs):
    if with_cnt:
        (t_hbm, gidx_hbm, sidx_hbm, z64_hbm, z8_hbm, ones_hbm,
         acc_hbm, cnt_hbm, gidx_v, sidx_v, rows_v, ones_v, acc_sh, cnt_sh,
         *gsems) = refs
    else:
        (t_hbm, gidx_hbm, sidx_hbm, z64_hbm,
         acc_hbm, gidx_v, sidx_v, rows_v, acc_sh, *gsems) = refs
    c = lax.axis_index("c")
    s = lax.axis_index("s")
    w = c * NS + s

    pltpu.sync_copy(gidx_hbm.at[w], gidx_v)
    pltpu.sync_copy(sidx_hbm.at[w], sidx_v)
    for b in range(NBUF):
        pltpu.async_copy(t_hbm.at[gidx_v.at[b]], rows_v.at[b], gsems[b])
    pltpu.sync_copy(z64_hbm, acc_sh.at[pl.ds(s * ROWS_PER_TILE, ROWS_PER_TILE)])
    if with_cnt:
        pltpu.sync_copy(z8_hbm, cnt_sh.at[pl.ds(s * ROWS_PER_TILE, ROWS_PER_TILE)])
        pltpu.sync_copy(ones_hbm, ones_v)
    plsc.subcore_barrier()

    @pl.loop(0, K_PER_W, step=NBUF)
    def _grp(g):
        for b in range(NBUF):
            j = g + b
            pltpu.make_async_copy(t_hbm.at[gidx_v.at[j]], rows_v.at[b],
                                  gsems[b]).wait()
            pltpu.sync_copy(rows_v.at[b], acc_sh.at[sidx_v.at[j]], add=True)
            if with_cnt:
                pltpu.sync_copy(ones_v, cnt_sh.at[sidx_v.at[j]], add=True)

            @pl.when(j + NBUF < K_PER_W)
            def _refill():
                pltpu.async_copy(t_hbm.at[gidx_v.at[j + NBUF]], rows_v.at[b],
                                 gsems[b])

    plsc.subcore_barrier()
    r = s // 8
    o = (s % 8) * ROWS_PER_TILE
    sl = pl.ds(s * ROWS_PER_TILE, ROWS_PER_TILE)
    pltpu.sync_copy(acc_sh.at[sl], acc_hbm.at[c, r, pl.ds(o, ROWS_PER_TILE)])
    if with_cnt:
        pltpu.sync_copy(cnt_sh.at[sl], cnt_hbm.at[c, r, pl.ds(o, ROWS_PER_TILE)])


_sc_scatter_cnt = pl.kernel(
    functools.partial(_sc_scatter_body, True),
    out_type=(
        jax.ShapeDtypeStruct((NC, NUM_REL, NP, HID_DIM), jnp.float32),
        jax.ShapeDtypeStruct((NC, NUM_REL, NP, 8), jnp.float32),
    ),
    mesh=_SC_MESH,
    scratch_types=[
        pltpu.VMEM((K_PER_W, EDGE_B), jnp.int32),
        pltpu.VMEM((K_PER_W, EDGE_B), jnp.int32),
        pltpu.VMEM((NBUF, EDGE_B, HID_DIM), jnp.float32),
        pltpu.VMEM((EDGE_B, 8), jnp.float32),
        pltpu.VMEM_SHARED((ACC_ROWS, HID_DIM), jnp.float32),
        pltpu.VMEM_SHARED((ACC_ROWS, 8), jnp.float32),
    ] + [pltpu.SemaphoreType.DMA] * NBUF,
    compiler_params=pltpu.CompilerParams(use_tc_tiling_on_sc=False),
)

_sc_scatter_nocnt = pl.kernel(
    functools.partial(_sc_scatter_body, False),
    out_type=jax.ShapeDtypeStruct((NC, NUM_REL, NP, HID_DIM), jnp.float32),
    mesh=_SC_MESH,
    scratch_types=[
        pltpu.VMEM((K_PER_W, EDGE_B), jnp.int32),
        pltpu.VMEM((K_PER_W, EDGE_B), jnp.int32),
        pltpu.VMEM((NBUF, EDGE_B, HID_DIM), jnp.float32),
        pltpu.VMEM_SHARED((ACC_ROWS, HID_DIM), jnp.float32),
    ] + [pltpu.SemaphoreType.DMA] * NBUF,
    compiler_params=pltpu.CompilerParams(use_tc_tiling_on_sc=False),
)


def _blockdiag(w):
    z = jnp.zeros_like(w)
    return jnp.concatenate([jnp.concatenate([w, z], axis=1),
                            jnp.concatenate([z, w], axis=1)], axis=0)


def kernel(x, edge_index, edge_type, W_rel1, W_root1, b1, ln1_g, ln1_b,
           W_rel2, W_root2, b2, ln2_g, ln2_b):
    x_pair = jnp.pad(x, ((0, NP - N_NODES), (0, 0))).reshape(NPH, 2 * IN_DIM)

    wrel1_bd = jnp.stack([_blockdiag(W_rel1[0]), _blockdiag(W_rel1[1])])
    wrel2_bd = jnp.stack([_blockdiag(W_rel2[0]), _blockdiag(W_rel2[1])])
    wroot1_bd = _blockdiag(W_root1)
    wroot2_bd = _blockdiag(W_root2)
    b1p = jnp.concatenate([b1, b1])
    b2p = jnp.concatenate([b2, b2])

    t1 = _table1_call(x_pair, wrel1_bd).reshape(ACC_ROWS, HID_DIM)
    gidx, sidx = _edge_call(edge_index, edge_type)

    z64 = jnp.zeros((ROWS_PER_TILE, HID_DIM), jnp.float32)
    z8 = jnp.zeros((ROWS_PER_TILE, 8), jnp.float32)
    ones8 = jnp.ones((EDGE_B, 8), jnp.float32)

    acc1, cnt = _sc_scatter_cnt(t1, gidx, sidx, z64, z8, ones8)
    acc1p = acc1.reshape(NC, NUM_REL, NPH, 128)
    cntp = cnt.reshape(NC, NUM_REL, NPH, 16)
    r1 = _root1_call(x_pair, wroot1_bd, b1p)

    h = _hid_call(r1, acc1p, cntp, ln1_g, ln1_b)
    t2 = _table2_call(h, wrel2_bd).reshape(ACC_ROWS, HID_DIM)

    acc2 = _sc_scatter_nocnt(t2, gidx, sidx, z64)
    acc2p = acc2.reshape(NC, NUM_REL, NPH, 128)
    r2 = _root2_call(h, wroot2_bd, b2p)

    out = _final_call(r2, acc2p, cntp, ln2_g, ln2_b)
    return out.reshape(N_NODES, HID_DIM)


# speedup vs baseline: 1.5345x; 1.0223x over previous
import functools

import jax
import jax.numpy as jnp
from jax import lax
from jax.experimental import pallas as pl
from jax.experimental.pallas import tpu as pltpu
from jax.experimental.pallas import tpu_sc as plsc

N_NODES = 10000
NP = 10240
NPH = NP // 2
N_EDGES = 320000
IN_DIM = 128
HID_DIM = 64
NUM_REL = 2

NC = 2
NS = 16
NW = NC * NS

EDGE_B = 128
E_PAD = 327680
K_PER_W = E_PAD // (NW * EDGE_B)
NBUF = 2
ACC_ROWS = NUM_REL * NP
ROWS_PER_TILE = ACC_ROWS // NS
_EB = N_EDGES // 128
_EBP = E_PAD // 128

NBH = 1024
_NG = NPH // NBH


def _table_body(pairup, x_ref, wbd_ref, t_ref):
    x = x_ref[...]
    if pairup:
        x = x.reshape(NBH, 2 * IN_DIM)
    t_ref[...] = jnp.dot(x, wbd_ref[0], preferred_element_type=jnp.float32)


def _make_table_call(pairup, in_dim, n_rows):
    blk = (2 * NBH, in_dim) if pairup else (NBH, 2 * in_dim)
    return pl.pallas_call(
        functools.partial(_table_body, pairup),
        grid=(NUM_REL, _NG),
        in_specs=[
            pl.BlockSpec(blk, lambda r, i: (i, 0)),
            pl.BlockSpec((1, 2 * in_dim, 128), lambda r, i: (r, 0, 0)),
        ],
        out_specs=pl.BlockSpec((NBH, 128), lambda r, i: (r * _NG + i, 0)),
        out_shape=jax.ShapeDtypeStruct((ACC_ROWS // 2, 128), jnp.float32),
    )


_table1_call = _make_table_call(True, IN_DIM, N_NODES)
_table2_call = _make_table_call(False, HID_DIM, NPH)


def _root_body(pairup, x_ref, wbd_ref, b_ref, r_ref):
    x = x_ref[...]
    if pairup:
        x = x.reshape(NBH, 2 * IN_DIM)
    r_ref[...] = (jnp.dot(x, wbd_ref[...], preferred_element_type=jnp.float32)
                  + b_ref[...][None, :])


def _make_root_call(pairup, in_dim):
    blk = (2 * NBH, in_dim) if pairup else (NBH, 2 * in_dim)
    return pl.pallas_call(
        functools.partial(_root_body, pairup),
        grid=(_NG,),
        in_specs=[
            pl.BlockSpec(blk, lambda i: (i, 0)),
            pl.BlockSpec((2 * in_dim, 128), lambda i: (0, 0)),
            pl.BlockSpec((128,), lambda i: (0,)),
        ],
        out_specs=pl.BlockSpec((NBH, 128), lambda i: (i, 0)),
        out_shape=jax.ShapeDtypeStruct((NPH, 128), jnp.float32),
    )


_root1_call = _make_root_call(True, IN_DIM)
_root2_call = _make_root_call(False, HID_DIM)


def _edge_body(ei_ref, et_ref, gidx_ref, sidx_ref):
    lanes = jax.lax.broadcasted_iota(jnp.int32, (_EBP - _EB, 128), 1)
    rows = jax.lax.broadcasted_iota(jnp.int32, (_EBP - _EB, 128), 0)
    pad = N_NODES + (rows * 128 + lanes) % (NP - N_NODES)
    et = et_ref[...].reshape(_EB, 128)
    src = ei_ref[0].reshape(_EB, 128)
    dst = ei_ref[1].reshape(_EB, 128)
    g = jnp.concatenate([et * NP + src, pad], axis=0)
    s = jnp.concatenate([et * NP + dst, pad], axis=0)
    gidx_ref[...] = g.reshape(NW, K_PER_W, EDGE_B)
    sidx_ref[...] = s.reshape(NW, K_PER_W, EDGE_B)


_edge_call = pl.pallas_call(
    _edge_body,
    out_shape=(
        jax.ShapeDtypeStruct((NW, K_PER_W, EDGE_B), jnp.int32),
        jax.ShapeDtypeStruct((NW, K_PER_W, EDGE_B), jnp.int32),
    ),
)


def _pair_divisor(cnt_ref, r, n):
    c_even = jnp.maximum(cnt_ref[0, r, :, 0:1] + cnt_ref[1, r, :, 0:1], 1.0)
    c_odd = jnp.maximum(cnt_ref[0, r, :, 8:9] + cnt_ref[1, r, :, 8:9], 1.0)
    return jnp.concatenate([jnp.broadcast_to(c_even, (n, HID_DIM)),
                            jnp.broadcast_to(c_odd, (n, HID_DIM))], axis=1)


def _combine(acc_ref, cnt_ref, n):
    return ((acc_ref[0, 0] + acc_ref[1, 0]) / _pair_divisor(cnt_ref, 0, n)
            + (acc_ref[0, 1] + acc_ref[1, 1]) / _pair_divisor(cnt_ref, 1, n))


def _ln_half(h, g, b):
    mu = jnp.mean(h, axis=-1, keepdims=True)
    var = jnp.mean((h - mu) ** 2, axis=-1, keepdims=True)
    return (h - mu) / jnp.sqrt(var + 1e-5) * g + b


def _layer_norm_pair(h, g, b):
    return jnp.concatenate([_ln_half(h[:, 0:HID_DIM], g, b),
                            _ln_half(h[:, HID_DIM:128], g, b)], axis=1)


def _hid_body(r1_ref, acc_ref, cnt_ref, g_ref, bln_ref, h_ref):
    h = r1_ref[...] + _combine(acc_ref, cnt_ref, NBH)
    h = _layer_norm_pair(h, g_ref[...][None, :], bln_ref[...][None, :])
    h_ref[...] = 0.5 * h * (1.0 + lax.erf(h * (2.0 ** -0.5)))


_acc_spec = pl.BlockSpec((NC, NUM_REL, NBH, 128), lambda *g: (0, 0, g[-1], 0))
_cnt_spec = pl.BlockSpec((NC, NUM_REL, NBH, 16), lambda *g: (0, 0, g[-1], 0))
_vec_spec = pl.BlockSpec((HID_DIM,), lambda *g: (0,))
_row_spec = pl.BlockSpec((NBH, 128), lambda *g: (g[-1], 0))

_hid_call = pl.pallas_call(
    _hid_body,
    grid=(_NG,),
    in_specs=[_row_spec, _acc_spec, _cnt_spec, _vec_spec, _vec_spec],
    out_specs=_row_spec,
    out_shape=jax.ShapeDtypeStruct((NPH, 128), jnp.float32),
)

_FNB = 1000


def _final_body(r2_ref, acc_ref, cnt_ref, g_ref, bln_ref, out_ref):
    h = r2_ref[...] + _combine(acc_ref, cnt_ref, _FNB)
    out_ref[...] = _layer_norm_pair(h, g_ref[...][None, :], bln_ref[...][None, :])


_final_call = pl.pallas_call(
    _final_body,
    grid=(N_NODES // (2 * _FNB),),
    in_specs=[
        pl.BlockSpec((_FNB, 128), lambda i: (i, 0)),
        pl.BlockSpec((NC, NUM_REL, _FNB, 128), lambda i: (0, 0, i, 0)),
        pl.BlockSpec((NC, NUM_REL, _FNB, 16), lambda i: (0, 0, i, 0)),
        _vec_spec,
        _vec_spec,
    ],
    out_specs=pl.BlockSpec((_FNB, 128), lambda i: (i, 0)),
    out_shape=jax.ShapeDtypeStruct((N_NODES // 2, 128), jnp.float32),
)


_SC_MESH = plsc.VectorSubcoreMesh(core_axis_name="c", subcore_axis_name="s")


def _sc_scatter_body(with_cnt, *refs):
    if with_cnt:
        (t_hbm, gidx_hbm, sidx_hbm, z64_hbm, z8_hbm, ones_hbm,
         acc_hbm, cnt_hbm, gidx_v, sidx_v, rows_v, ones_v, acc_sh, cnt_sh,
         *gsems) = refs
    else:
        (t_hbm, gidx_hbm, sidx_hbm, z64_hbm,
         acc_hbm, gidx_v, sidx_v, rows_v, acc_sh, *gsems) = refs
    c = lax.axis_index("c")
    s = lax.axis_index("s")
    w = c * NS + s

    pltpu.sync_copy(gidx_hbm.at[w], gidx_v)
    pltpu.sync_copy(sidx_hbm.at[w], sidx_v)
    for b in range(NBUF):
        pltpu.async_copy(t_hbm.at[gidx_v.at[b]], rows_v.at[b], gsems[b])
    pltpu.sync_copy(z64_hbm, acc_sh.at[pl.ds(s * ROWS_PER_TILE, ROWS_PER_TILE)])
    if with_cnt:
        pltpu.sync_copy(z8_hbm, cnt_sh.at[pl.ds(s * ROWS_PER_TILE, ROWS_PER_TILE)])
        pltpu.sync_copy(ones_hbm, ones_v)
    plsc.subcore_barrier()

    @pl.loop(0, K_PER_W, step=NBUF)
    def _grp(g):
        for b in range(NBUF):
            j = g + b
            pltpu.make_async_copy(t_hbm.at[gidx_v.at[j]], rows_v.at[b],
                                  gsems[b]).wait()
            pltpu.sync_copy(rows_v.at[b], acc_sh.at[sidx_v.at[j]], add=True)
            if with_cnt:
                pltpu.sync_copy(ones_v, cnt_sh.at[sidx_v.at[j]], add=True)

            @pl.when(j + NBUF < K_PER_W)
            def _refill():
                pltpu.async_copy(t_hbm.at[gidx_v.at[j + NBUF]], rows_v.at[b],
                                 gsems[b])

    plsc.subcore_barrier()
    r = s // 8
    o = (s % 8) * ROWS_PER_TILE
    sl = pl.ds(s * ROWS_PER_TILE, ROWS_PER_TILE)
    pltpu.sync_copy(acc_sh.at[sl], acc_hbm.at[c, r, pl.ds(o, ROWS_PER_TILE)])
    if with_cnt:
        pltpu.sync_copy(cnt_sh.at[sl], cnt_hbm.at[c, r, pl.ds(o, ROWS_PER_TILE)])


_sc_scatter_cnt = pl.kernel(
    functools.partial(_sc_scatter_body, True),
    out_type=(
        jax.ShapeDtypeStruct((NC, NUM_REL, NP, HID_DIM), jnp.float32),
        jax.ShapeDtypeStruct((NC, NUM_REL, NP, 8), jnp.float32),
    ),
    mesh=_SC_MESH,
    scratch_types=[
        pltpu.VMEM((K_PER_W, EDGE_B), jnp.int32),
        pltpu.VMEM((K_PER_W, EDGE_B), jnp.int32),
        pltpu.VMEM((NBUF, EDGE_B, HID_DIM), jnp.float32),
        pltpu.VMEM((EDGE_B, 8), jnp.float32),
        pltpu.VMEM_SHARED((ACC_ROWS, HID_DIM), jnp.float32),
        pltpu.VMEM_SHARED((ACC_ROWS, 8), jnp.float32),
    ] + [pltpu.SemaphoreType.DMA] * NBUF,
    compiler_params=pltpu.CompilerParams(use_tc_tiling_on_sc=False),
)

_sc_scatter_nocnt = pl.kernel(
    functools.partial(_sc_scatter_body, False),
    out_type=jax.ShapeDtypeStruct((NC, NUM_REL, NP, HID_DIM), jnp.float32),
    mesh=_SC_MESH,
    scratch_types=[
        pltpu.VMEM((K_PER_W, EDGE_B), jnp.int32),
        pltpu.VMEM((K_PER_W, EDGE_B), jnp.int32),
        pltpu.VMEM((NBUF, EDGE_B, HID_DIM), jnp.float32),
        pltpu.VMEM_SHARED((ACC_ROWS, HID_DIM), jnp.float32),
    ] + [pltpu.SemaphoreType.DMA] * NBUF,
    compiler_params=pltpu.CompilerParams(use_tc_tiling_on_sc=False),
)


def _blockdiag(w):
    z = jnp.zeros_like(w)
    return jnp.concatenate([jnp.concatenate([w, z], axis=1),
                            jnp.concatenate([z, w], axis=1)], axis=0)


def kernel(x, edge_index, edge_type, W_rel1, W_root1, b1, ln1_g, ln1_b,
           W_rel2, W_root2, b2, ln2_g, ln2_b):
    wrel1_bd = jnp.stack([_blockdiag(W_rel1[0]), _blockdiag(W_rel1[1])])
    wrel2_bd = jnp.stack([_blockdiag(W_rel2[0]), _blockdiag(W_rel2[1])])
    wroot1_bd = _blockdiag(W_root1)
    wroot2_bd = _blockdiag(W_root2)
    b1p = jnp.concatenate([b1, b1])
    b2p = jnp.concatenate([b2, b2])

    t1 = _table1_call(x, wrel1_bd).reshape(ACC_ROWS, HID_DIM)
    gidx, sidx = _edge_call(edge_index, edge_type)

    z64 = jnp.zeros((ROWS_PER_TILE, HID_DIM), jnp.float32)
    z8 = jnp.zeros((ROWS_PER_TILE, 8), jnp.float32)
    ones8 = jnp.ones((EDGE_B, 8), jnp.float32)

    acc1, cnt = _sc_scatter_cnt(t1, gidx, sidx, z64, z8, ones8)
    acc1p = acc1.reshape(NC, NUM_REL, NPH, 128)
    cntp = cnt.reshape(NC, NUM_REL, NPH, 16)
    r1 = _root1_call(x, wroot1_bd, b1p)

    h = _hid_call(r1, acc1p, cntp, ln1_g, ln1_b)
    t2 = _table2_call(h, wrel2_bd).reshape(ACC_ROWS, HID_DIM)

    acc2 = _sc_scatter_nocnt(t2, gidx, sidx, z64)
    acc2p = acc2.reshape(NC, NUM_REL, NPH, 128)
    r2 = _root2_call(h, wroot2_bd, b2p)

    out = _final_call(r2, acc2p, cntp, ln2_g, ln2_b)
    return out.reshape(N_NODES, HID_DIM)
